# Initial kernel scaffold; baseline (speedup 1.0000x reference)
import jax
import jax.numpy as jnp
from jax.experimental import pallas as pl


def kernel(occ_so, phi_ref):
    raise NotImplementedError("write your pallas kernel here")


# speedup vs baseline: 1.2565x; 1.2565x over previous
import functools

import jax
import jax.numpy as jnp
from jax import lax
from jax.experimental import pallas as pl
from jax.experimental.pallas import tpu as pltpu
from jax.experimental.pallas import tpu_sc as plsc

N_DET = 16
N_SO = 512
N_E = 32
BATCH = 4096

_info = plsc.get_sparse_core_info()
NC, NS, L = _info.num_cores, _info.num_subcores, _info.num_lanes
NW = NC * NS

B_PER_W = BATCH // NW
NB = 4
N_CHUNKS = B_PER_W // NB
ROWS_PER_CHUNK = NB * N_DET * N_E
IDX_ROWS = ROWS_PER_CHUNK // 128


def _spo_body(occ_hbm, tab_hbm, out_hbm, occ_v, idx_v, buf_v, sem):
    wid = lax.axis_index("s") * NC + lax.axis_index("c")

    pltpu.sync_copy(occ_hbm.at[pl.ds(wid * (B_PER_W * N_E), B_PER_W * N_E)],
                    occ_v)

    def chunk(g, carry):
        for ii in range(NB):
            o_base = (g * NB + ii) * N_E
            o0 = occ_v[pl.ds(o_base, L)]
            o1 = occ_v[pl.ds(o_base + L, L)]
            for d in range(N_DET):
                row = ii * N_DET + d
                r = row // 4
                c = (row % 4) * N_E
                idx_v[r, pl.ds(c, L)] = o0 + d * N_SO
                idx_v[r, pl.ds(c + L, L)] = o1 + d * N_SO

        copies = []
        for j in range(IDX_ROWS):
            copies.append(
                pltpu.async_copy(tab_hbm.at[idx_v.at[j]],
                                 buf_v.at[pl.ds(j * 128, 128)], sem))
        for cp in copies:
            cp.wait()

        out_base = wid * (B_PER_W * N_DET * N_E) + g * ROWS_PER_CHUNK
        pltpu.sync_copy(buf_v, out_hbm.at[pl.ds(out_base, ROWS_PER_CHUNK)])
        return carry

    lax.fori_loop(0, N_CHUNKS, chunk, 0)


@functools.partial(jax.jit, static_argnames=())
def kernel(occ_so, phi_ref):
    occ_flat = occ_so.astype(jnp.int32).reshape(BATCH * N_E)
    tab = phi_ref.reshape(N_DET * N_SO, N_E)

    mesh = plsc.VectorSubcoreMesh(core_axis_name="c", subcore_axis_name="s")
    out_flat = pl.kernel(
        _spo_body,
        mesh=mesh,
        compiler_params=pltpu.CompilerParams(use_tc_tiling_on_sc=False),
        out_type=jax.ShapeDtypeStruct((BATCH * N_DET * N_E, N_E), jnp.float32),
        scratch_types=[
            pltpu.VMEM((B_PER_W * N_E,), jnp.int32),
            pltpu.VMEM((IDX_ROWS, 128), jnp.int32),
            pltpu.VMEM((ROWS_PER_CHUNK, N_E), jnp.float32),
            pltpu.SemaphoreType.DMA,
        ],
    )(occ_flat, tab)
    return out_flat.reshape(BATCH, N_DET, N_E, N_E)


# speedup vs baseline: 1.2573x; 1.0006x over previous
import functools

import jax
import jax.numpy as jnp
from jax import lax
from jax.experimental import pallas as pl
from jax.experimental.pallas import tpu as pltpu
from jax.experimental.pallas import tpu_sc as plsc

N_DET = 16
N_SO = 512
N_E = 32
BATCH = 4096

_info = plsc.get_sparse_core_info()
NC, NS, L = _info.num_cores, _info.num_subcores, _info.num_lanes
NW = NC * NS

B_PER_W = BATCH // NW
NB = 2
N_CHUNKS = B_PER_W // NB
ROWS_PER_CHUNK = NB * N_DET * N_E
IDX_ROWS = ROWS_PER_CHUNK // 128
OUT_PER_W = B_PER_W * N_DET * N_E


def _spo_body(occ_hbm, tab_hbm, out_hbm,
              occ_v, idx_a, idx_b, buf_a, buf_b,
              gsem_a, gsem_b, wsem_a, wsem_b):
    wid = lax.axis_index("s") * NC + lax.axis_index("c")
    out_base = wid * OUT_PER_W

    pltpu.sync_copy(occ_hbm.at[pl.ds(wid * (B_PER_W * N_E), B_PER_W * N_E)],
                    occ_v)

    def expand_idx(g, idx_v):
        for ii in range(NB):
            o_base = (g * NB + ii) * N_E
            o0 = occ_v[pl.ds(o_base, L)]
            o1 = occ_v[pl.ds(o_base + L, L)]
            for d in range(N_DET):
                row = ii * N_DET + d
                r = row // 4
                c = (row % 4) * N_E
                idx_v[r, pl.ds(c, L)] = o0 + d * N_SO
                idx_v[r, pl.ds(c + L, L)] = o1 + d * N_SO

    def fire_gathers(idx_v, buf_v, sem):
        for j in range(IDX_ROWS):
            pltpu.async_copy(tab_hbm.at[idx_v.at[j]],
                             buf_v.at[pl.ds(j * 128, 128)], sem)

    def drain_gathers(idx_v, buf_v, sem):
        for j in range(IDX_ROWS):
            pltpu.make_async_copy(tab_hbm.at[idx_v.at[j]],
                                  buf_v.at[pl.ds(j * 128, 128)], sem).wait()

    def fire_writeback(buf_v, g, sem):
        pltpu.async_copy(buf_v, out_hbm.at[pl.ds(out_base + g * ROWS_PER_CHUNK,
                                                 ROWS_PER_CHUNK)], sem)

    def drain_writeback(buf_v, sem):
        pltpu.make_async_copy(buf_v, out_hbm.at[pl.ds(out_base,
                                                      ROWS_PER_CHUNK)],
                              sem).wait()

    expand_idx(0, idx_a)
    fire_gathers(idx_a, buf_a, gsem_a)

    def pair(h, carry):
        ga = 2 * h
        gb = 2 * h + 1

        @pl.when(h > 0)
        def _():
            drain_writeback(buf_b, wsem_b)
        expand_idx(gb, idx_b)
        fire_gathers(idx_b, buf_b, gsem_b)

        drain_gathers(idx_a, buf_a, gsem_a)
        fire_writeback(buf_a, ga, wsem_a)

        @pl.when(h < N_CHUNKS // 2 - 1)
        def _():
            drain_writeback(buf_a, wsem_a)
            expand_idx(ga + 2, idx_a)
            fire_gathers(idx_a, buf_a, gsem_a)

        drain_gathers(idx_b, buf_b, gsem_b)
        fire_writeback(buf_b, gb, wsem_b)
        return carry

    lax.fori_loop(0, N_CHUNKS // 2, pair, 0)
    drain_writeback(buf_a, wsem_a)
    drain_writeback(buf_b, wsem_b)


@functools.partial(jax.jit, static_argnames=())
def kernel(occ_so, phi_ref):
    occ_flat = occ_so.astype(jnp.int32).reshape(BATCH * N_E)
    tab = phi_ref.reshape(N_DET * N_SO, N_E)

    mesh = plsc.VectorSubcoreMesh(core_axis_name="c", subcore_axis_name="s")
    out_flat = pl.kernel(
        _spo_body,
        mesh=mesh,
        compiler_params=pltpu.CompilerParams(use_tc_tiling_on_sc=False),
        out_type=jax.ShapeDtypeStruct((BATCH * N_DET * N_E, N_E), jnp.float32),
        scratch_types=[
            pltpu.VMEM((B_PER_W * N_E,), jnp.int32),
            pltpu.VMEM((IDX_ROWS, 128), jnp.int32),
            pltpu.VMEM((IDX_ROWS, 128), jnp.int32),
            pltpu.VMEM((ROWS_PER_CHUNK, N_E), jnp.float32),
            pltpu.VMEM((ROWS_PER_CHUNK, N_E), jnp.float32),
            pltpu.SemaphoreType.DMA,
            pltpu.SemaphoreType.DMA,
            pltpu.SemaphoreType.DMA,
            pltpu.SemaphoreType.DMA,
        ],
    )(occ_flat, tab)
    return out_flat.reshape(BATCH, N_DET, N_E, N_E)


# speedup vs baseline: 1.6467x; 1.3097x over previous
import functools

import jax
import jax.numpy as jnp
from jax import lax
from jax.experimental import pallas as pl
from jax.experimental.pallas import tpu as pltpu
from jax.experimental.pallas import tpu_sc as plsc

N_DET = 16
N_SO = 512
N_E = 32
BATCH = 4096

_info = plsc.get_sparse_core_info()
NC, NS, L = _info.num_cores, _info.num_subcores, _info.num_lanes
NW = NC * NS

BH = BATCH // 2
JT = 8
N_JT = N_E // JT
N_STEPS = N_E * N_JT


def _spo_body(occ_hbm, tab_hbm, out_hbm,
              occ_v, tab_v, buf_a, buf_b, wsem_a, wsem_b):
    wid = lax.axis_index("s") * NC + lax.axis_index("c")
    d = wid // 2
    h = wid % 2
    b0 = h * BH

    pltpu.sync_copy(tab_hbm.at[d], tab_v)
    pltpu.sync_copy(occ_hbm.at[:, pl.ds(b0, BH)], occ_v)

    def produce(e, jt, buf):
        jbase = jt * JT

        def bg_body(g, carry):
            o = occ_v[e, pl.ds(g * L, L)]
            base = o * N_E + jbase
            for j in range(JT):
                buf[j, pl.ds(g * L, L)] = plsc.load_gather(tab_v, [base + j])
            return carry

        lax.fori_loop(0, BH // L, bg_body, 0)

    def fire_wb(e, jt, buf, sem):
        pltpu.async_copy(buf,
                         out_hbm.at[d, e, pl.ds(jt * JT, JT), pl.ds(b0, BH)],
                         sem)

    def drain_wb(buf, sem):
        pltpu.make_async_copy(buf,
                              out_hbm.at[d, 0, pl.ds(0, JT), pl.ds(b0, BH)],
                              sem).wait()

    def pair(p, carry):
        ta = 2 * p
        tb = ta + 1
        ea, jta = ta // N_JT, ta % N_JT
        eb, jtb = tb // N_JT, tb % N_JT

        @pl.when(p > 0)
        def _():
            drain_wb(buf_a, wsem_a)
        produce(ea, jta, buf_a)
        fire_wb(ea, jta, buf_a, wsem_a)

        @pl.when(p > 0)
        def _():
            drain_wb(buf_b, wsem_b)
        produce(eb, jtb, buf_b)
        fire_wb(eb, jtb, buf_b, wsem_b)
        return carry

    lax.fori_loop(0, N_STEPS // 2, pair, 0)
    drain_wb(buf_a, wsem_a)
    drain_wb(buf_b, wsem_b)


@functools.partial(jax.jit, static_argnames=())
def kernel(occ_so, phi_ref):
    occ_t = occ_so.astype(jnp.int32).T
    tab = phi_ref.reshape(N_DET, N_SO * N_E)

    mesh = plsc.VectorSubcoreMesh(core_axis_name="c", subcore_axis_name="s")
    out_phys = pl.kernel(
        _spo_body,
        mesh=mesh,
        compiler_params=pltpu.CompilerParams(needs_layout_passes=False),
        out_type=jax.ShapeDtypeStruct((N_DET, N_E, N_E, BATCH), jnp.float32),
        scratch_types=[
            pltpu.VMEM((N_E, BH), jnp.int32),
            pltpu.VMEM((N_SO * N_E,), jnp.float32),
            pltpu.VMEM((JT, BH), jnp.float32),
            pltpu.VMEM((JT, BH), jnp.float32),
            pltpu.SemaphoreType.DMA,
            pltpu.SemaphoreType.DMA,
        ],
    )(occ_t, tab)
    return jnp.transpose(out_phys, (3, 0, 1, 2))


# speedup vs baseline: 2.8162x; 1.7102x over previous
import functools

import jax
import jax.numpy as jnp
from jax import lax
from jax.experimental import pallas as pl
from jax.experimental.pallas import tpu as pltpu
from jax.experimental.pallas import tpu_sc as plsc

N_DET = 16
N_SO = 512
N_E = 32
BATCH = 4096

_info = plsc.get_sparse_core_info()
NC, NS, L = _info.num_cores, _info.num_subcores, _info.num_lanes
NW = NC * NS

BH = BATCH // 2
JT = 8
N_JT = N_E // JT
N_STEPS = N_E * N_JT


def _spo_body(occ_hbm, tab_hbm, out_hbm,
              occ_v, tab_v, buf_a, buf_b, wsem_a, wsem_b):
    wid = lax.axis_index("s") * NC + lax.axis_index("c")
    d = wid // 2
    h = wid % 2
    b0 = h * BH

    pltpu.sync_copy(tab_hbm.at[d], tab_v)
    pltpu.sync_copy(occ_hbm.at[:, pl.ds(b0, BH)], occ_v)

    def produce(e, jt, buf):
        jbase = jt * JT

        @plsc.parallel_loop(0, BH // L, unroll=4)
        def bg_body(g):
            o = occ_v[e, pl.ds(g * L, L)]
            base = o * N_E + jbase
            for j in range(JT):
                buf[j, pl.ds(g * L, L)] = plsc.load_gather(tab_v, [base + j])

    def fire_wb(e, jt, buf, sem):
        pltpu.async_copy(buf,
                         out_hbm.at[d, e, pl.ds(jt * JT, JT), pl.ds(b0, BH)],
                         sem)

    def drain_wb(buf, sem):
        pltpu.make_async_copy(buf,
                              out_hbm.at[d, 0, pl.ds(0, JT), pl.ds(b0, BH)],
                              sem).wait()

    def pair(p, carry):
        ta = 2 * p
        tb = ta + 1
        ea, jta = ta // N_JT, ta % N_JT
        eb, jtb = tb // N_JT, tb % N_JT

        @pl.when(p > 0)
        def _():
            drain_wb(buf_a, wsem_a)
        produce(ea, jta, buf_a)
        fire_wb(ea, jta, buf_a, wsem_a)

        @pl.when(p > 0)
        def _():
            drain_wb(buf_b, wsem_b)
        produce(eb, jtb, buf_b)
        fire_wb(eb, jtb, buf_b, wsem_b)
        return carry

    lax.fori_loop(0, N_STEPS // 2, pair, 0)
    drain_wb(buf_a, wsem_a)
    drain_wb(buf_b, wsem_b)


@functools.partial(jax.jit, static_argnames=())
def kernel(occ_so, phi_ref):
    occ_t = occ_so.astype(jnp.int32).T
    tab = phi_ref.reshape(N_DET, N_SO * N_E)

    mesh = plsc.VectorSubcoreMesh(core_axis_name="c", subcore_axis_name="s")
    out_phys = pl.kernel(
        _spo_body,
        mesh=mesh,
        compiler_params=pltpu.CompilerParams(needs_layout_passes=False),
        out_type=jax.ShapeDtypeStruct((N_DET, N_E, N_E, BATCH), jnp.float32),
        scratch_types=[
            pltpu.VMEM((N_E, BH), jnp.int32),
            pltpu.VMEM((N_SO * N_E,), jnp.float32),
            pltpu.VMEM((JT, BH), jnp.float32),
            pltpu.VMEM((JT, BH), jnp.float32),
            pltpu.SemaphoreType.DMA,
            pltpu.SemaphoreType.DMA,
        ],
    )(occ_t, tab)
    return jnp.transpose(out_phys, (3, 0, 1, 2))
